# Initial kernel scaffold; baseline (speedup 1.0000x reference)
import jax
import jax.numpy as jnp
from jax.experimental import pallas as pl


def kernel(preds, labels):
    raise NotImplementedError("write your pallas kernel here")


# speedup vs baseline: 19.0559x; 19.0559x over previous
import functools

import jax
import jax.numpy as jnp
from jax import lax
from jax.experimental import pallas as pl
from jax.experimental.pallas import tpu as pltpu
from jax.experimental.pallas import tpu_sc as plsc

N = 131072
C = 20
K = 2048
NTILES = 32
PT = N // NTILES
SUB = 512
NSUB = PT // SUB


def _sc_body(preds_hbm, labels_hbm, out_hbm, pbuf, lbuf, hist):
    wid = lax.axis_index("c") * 16 + lax.axis_index("s")

    zeros16 = jnp.zeros((16,), jnp.float32)
    ones16 = jnp.ones((16,), jnp.float32)

    def _zero(i, carry):
        hist[pl.ds(i * 16, 16)] = zeros16
        return carry
    lax.fori_loop(0, 2 * C * K // 16, _zero, 0)

    lane = lax.iota(jnp.int32, 16)

    def _group(g, carry):
        p0 = g * 16
        lab = lbuf[pl.ds(p0, 16)]
        rbase = (p0 + lane) * C
        v = [plsc.load_gather(pbuf, [rbase + c]) for c in range(C)]
        m = v[0]
        for c in range(1, C):
            m = jnp.maximum(m, v[c])
        t = [jnp.exp(v[c] - m) for c in range(C)]
        s = t[0]
        for c in range(1, C):
            s = s + t[c]
        rinv = 1.0 / s
        fgbin = jnp.zeros((16,), jnp.int32)
        for c in range(C):
            p = t[c] * rinv
            err = jnp.abs(1.0 - p)
            b = (err * K).astype(jnp.int32)
            b = jnp.minimum(jnp.maximum(b, 0), K - 1)
            plsc.addupdate_scatter(hist, [b + c * K], ones16)
            fgbin = jnp.where(lab == c, b, fgbin)
        plsc.addupdate_scatter(hist, [(lab + C) * K + fgbin], ones16)
        return carry

    for sidx in range(NSUB):
        base = wid * PT + sidx * SUB
        pltpu.sync_copy(preds_hbm.at[pl.ds(base * C, SUB * C)], pbuf)
        pltpu.sync_copy(labels_hbm.at[pl.ds(base, SUB)], lbuf)
        lax.fori_loop(0, SUB // 16, _group, 0)

    for r in range(2 * C):
        pltpu.sync_copy(hist.at[pl.ds(r * K, K)], out_hbm.at[wid, r])


@functools.partial(
    pl.kernel,
    out_type=jax.ShapeDtypeStruct((NTILES, 2 * C, K), jnp.float32),
    mesh=plsc.VectorSubcoreMesh(core_axis_name="c", subcore_axis_name="s"),
    compiler_params=pltpu.CompilerParams(needs_layout_passes=False),
    scratch_types=[
        pltpu.VMEM((SUB * C,), jnp.float32),
        pltpu.VMEM((SUB,), jnp.int32),
        pltpu.VMEM((2 * C * K,), jnp.float32),
    ],
)
def _sc_hist(preds_hbm, labels_hbm, out_hbm, pbuf, lbuf, hist):
    _sc_body(preds_hbm, labels_hbm, out_hbm, pbuf, lbuf, hist)


def _tc_body(hist_ref, out_ref):
    tot = jnp.sum(hist_ref[...], axis=0)
    cnt = tot[:C, :]
    fg = tot[C:, :]
    ir = lax.broadcasted_iota(jnp.int32, (K, K), 0)
    ic = lax.broadcasted_iota(jnp.int32, (K, K), 1)
    M = (ir >= ic).astype(jnp.float32)
    dn = (((1,), (0,)), ((), ()))
    Nk = lax.dot_general(cnt, M, dn, preferred_element_type=jnp.float32)
    Fk = lax.dot_general(fg, M, dn, preferred_element_type=jnp.float32)
    G = Fk[:, 0:1]
    denom = G + Nk - Fk
    I = jnp.where(denom > 0, 1.0 - (G - Fk) / denom, 0.0)
    loss_c = (jnp.sum(I, axis=1, keepdims=True) - 0.5 * I[:, 0:1]) * (1.0 / K)
    present = (G > 0).astype(jnp.float32)
    loss = jnp.sum(loss_c * present) / jnp.maximum(jnp.sum(present), 1.0)
    out_ref[...] = jnp.broadcast_to(loss, (1, 1))


def _tc_finish(hist):
    return pl.pallas_call(
        _tc_body,
        out_shape=jax.ShapeDtypeStruct((1, 1), jnp.float32),
    )(hist)


def kernel(preds, labels):
    labels = labels.astype(jnp.int32)
    hist = _sc_hist(preds.reshape(-1), labels)
    return _tc_finish(hist)[0, 0]
